# baseline (speedup 1.0000x reference)
import functools

import jax
import jax.numpy as jnp
from jax import lax
from jax.experimental import pallas as pl
from jax.experimental.pallas import tpu as pltpu
from jax.experimental.pallas import tpu_sc as plsc

_B, _N, _E, _MC = 8, 1024, 128, 16
_BN = _B * _N
_C1, _C2, _LBL = 240, 120, 104
_NW = 32
_NGRP = 16
_NCB = 2
_CB = _E // _NCB
_NODES_W = _BN // _NGRP
_NCHK = 256
_L = 16
_NV = _CB // _L


def _sc_body(table_hbm, ch_hbm, xs_hbm, xj_hbm, table_v, ch_v, xs_v, xj_v):
    wid = lax.axis_index("s") * 2 + lax.axis_index("c")
    ng = wid // _NCB
    cb = wid % _NCB
    pltpu.sync_copy(table_hbm.at[:, pl.ds(cb * _CB, _CB)], table_v)
    pltpu.sync_copy(ch_hbm.at[:, pl.ds(ng * _NODES_W, _NODES_W)], ch_v)
    zero = jnp.zeros((_L,), jnp.float32)
    for v in range(_NV):
        table_v[0, pl.ds(v * _L, _L)] = zero

    def chunk_body(c, carry):
        def group_body(g, carry2):
            base = c * _NCHK + g * _L
            sbase = g * _L
            cvs = [ch_v[j, pl.ds(base, _L)] for j in range(_MC)]
            for n in range(_L):
                ch = cvs[_MC - 1][n]
                run = [table_v[ch, pl.ds(v * _L, _L)] for v in range(_NV)]
                xj = list(run)
                for j in range(_MC - 2, 0, -1):
                    ch = cvs[j][n]
                    for v in range(_NV):
                        run[v] = run[v] + table_v[ch, pl.ds(v * _L, _L)]
                        xj[v] = xj[v] + run[v]
                ch = cvs[0][n]
                for v in range(_NV):
                    run[v] = run[v] + table_v[ch, pl.ds(v * _L, _L)]
                    xs_v[sbase + n, pl.ds(v * _L, _L)] = run[v]
                    xj_v[sbase + n, pl.ds(v * _L, _L)] = xj[v]
            return carry2

        lax.fori_loop(0, _NCHK // _L, group_body, 0)
        pltpu.sync_copy(xs_v, xs_hbm.at[pl.ds(ng * _NODES_W + c * _NCHK, _NCHK),
                                        pl.ds(cb * _CB, _CB)])
        pltpu.sync_copy(xj_v, xj_hbm.at[pl.ds(ng * _NODES_W + c * _NCHK, _NCHK),
                                        pl.ds(cb * _CB, _CB)])
        return carry

    lax.fori_loop(0, _NODES_W // _NCHK, chunk_body, 0)


@jax.jit
def _sc_gather(table, ch_t):
    mesh = plsc.VectorSubcoreMesh(core_axis_name="c", subcore_axis_name="s")
    f = pl.kernel(
        _sc_body,
        out_type=[jax.ShapeDtypeStruct((_BN, _E), jnp.float32),
                  jax.ShapeDtypeStruct((_BN, _E), jnp.float32)],
        mesh=mesh,
        scratch_types=[
            pltpu.VMEM((_N, _CB), jnp.float32),
            pltpu.VMEM((_MC, _NODES_W), jnp.int32),
            pltpu.VMEM((_NCHK, _CB), jnp.float32),
            pltpu.VMEM((_NCHK, _CB), jnp.float32),
        ],
        compiler_params=pltpu.CompilerParams(use_tc_tiling_on_sc=False,
                                             needs_layout_passes=False),
    )
    return f(table, ch_t)


def _tc_body(nodes_ref, xs_ref, xj_ref, ch_ref, a_ref, b1_ref,
             w2_ref, b2_ref, g_ref, bfc_ref, out_ref):
    ch = ch_ref[...]
    ns = jnp.sum((ch != 0).astype(jnp.float32), axis=1, keepdims=True)
    is1 = ns == 1.0
    alpha = jnp.where(is1 & (ch[:, 0:1] != 0), 0.5, 0.0)
    beta = jnp.where(is1, 0.0, 1.0 / (ns - 1.0))
    xs = xs_ref[...]
    xr = alpha * xs + beta * xj_ref[...]
    xl = xs - xr
    x = jnp.dot(nodes_ref[...], a_ref[0], preferred_element_type=jnp.float32)
    x = x + jnp.dot(xr, a_ref[1], preferred_element_type=jnp.float32)
    x = x + jnp.dot(xl, a_ref[2], preferred_element_type=jnp.float32)
    h1 = jnp.tanh(x + b1_ref[...])
    h2 = jnp.tanh(
        lax.dot_general(h1, w2_ref[...], (((1,), (1,)), ((), ())),
                        preferred_element_type=jnp.float32) + b2_ref[...])
    m8 = jnp.max(h2.reshape(8, _N // 8, _C2), axis=1)
    m4 = jnp.max(m8.reshape(4, 2, _C2), axis=1)
    m2 = jnp.max(m4.reshape(2, 2, _C2), axis=1)
    m1 = jnp.max(m2, axis=0, keepdims=True)
    p = jnp.concatenate([m1, m2, m4, m8], axis=0)
    o = jnp.sum(p[:, :, None] * g_ref[...], axis=(0, 1)) + bfc_ref[0]
    out_ref[pl.ds(pl.program_id(0), 1), :] = o[None, :]


@functools.partial(jax.jit, static_argnames=())
def _tc_dense(nodes_f, xs_t, xj_t, ch_f, a, b1, w2, b2, g, bfc):
    full = lambda shape: pl.BlockSpec(shape, lambda b: (0,) * len(shape))
    return pl.pallas_call(
        _tc_body,
        grid=(_B,),
        in_specs=[
            pl.BlockSpec((_N, _E), lambda b: (b, 0)),
            pl.BlockSpec((_N, _E), lambda b: (b, 0)),
            pl.BlockSpec((_N, _E), lambda b: (b, 0)),
            pl.BlockSpec((_N, _MC), lambda b: (b, 0)),
            full((3, _E, _C1)),
            full((1, _C1)), full((_C2, _C1)), full((1, _C2)),
            full((15, _C2, _LBL)), full((1, _LBL)),
        ],
        out_specs=pl.BlockSpec((_B, _LBL), lambda b: (0, 0)),
        out_shape=jax.ShapeDtypeStruct((_B, _LBL), jnp.float32),
    )(nodes_f, xs_t, xj_t, ch_f, a, b1, w2, b2, g, bfc)


def kernel(nodes, children, W1, b1, W2, b2, Wfc, bfc):
    ch_f = children.reshape(_BN, _MC)
    xs_t, xj_t = _sc_gather(nodes[0], ch_f.T)

    a = W1.reshape(_C1, _E, 3).transpose(2, 1, 0)
    g1 = Wfc[:, 0:120].reshape(_LBL, _C2, 1).transpose(2, 1, 0)
    g2 = Wfc[:, 120:360].reshape(_LBL, _C2, 2).transpose(2, 1, 0)
    g3 = Wfc[:, 360:840].reshape(_LBL, _C2, 4).transpose(2, 1, 0)
    g4 = Wfc[:, 840:1800].reshape(_LBL, _C2, 8).transpose(2, 1, 0)
    g = jnp.concatenate([g1, g2, g3, g4], axis=0)
    return _tc_dense(nodes.reshape(_BN, _E), xs_t, xj_t, ch_f, a,
                     b1[None, :], W2, b2[None, :], g, bfc[None, :])


# speedup vs baseline: 1.4905x; 1.4905x over previous
import functools

import jax
import jax.numpy as jnp
from jax import lax
from jax.experimental import pallas as pl
from jax.experimental.pallas import tpu as pltpu
from jax.experimental.pallas import tpu_sc as plsc

_B, _N, _E, _MC = 8, 1024, 128, 16
_BN = _B * _N
_C1, _C2, _LBL = 240, 120, 104
_NW = 32
_NGRP = 8
_NCB = 4
_CB = _E // _NCB
_NODES_W = _BN // _NGRP
_L = 16
_UN = 16


def _sc_body(table_hbm, ch_hbm, xs_hbm, xj_hbm, table_v, ch_v, xs_v, xj_v):
    wid = lax.axis_index("s") * 2 + lax.axis_index("c")
    ng = wid // _NCB
    cb = wid % _NCB
    pltpu.sync_copy(table_hbm.at[:, pl.ds(cb * _CB, _CB)], table_v)
    pltpu.sync_copy(ch_hbm.at[:, pl.ds(ng * _NODES_W, _NODES_W)], ch_v)
    zero = jnp.zeros((_L,), jnp.float32)
    table_v[0, pl.ds(0, _L)] = zero
    table_v[0, pl.ds(_L, _L)] = zero

    one = jnp.full((_L,), 1.0, jnp.float32)
    izero = jnp.zeros((_L,), jnp.int32)

    def group_body(g, carry):
        base = g * _L
        cvs = [ch_v[j, pl.ds(base, _L)] for j in range(_MC)]
        ns = zero
        for j in range(_MC):
            ns = ns + jnp.where(cvs[j] != izero, one, zero)
        recv = one / (ns - one)
        m1v = jnp.where(ns == one, one, zero)
        for n in range(_L):
            node = base + n
            rec = recv[n]
            m1 = m1v[n]
            ch = cvs[_MC - 1][n]
            run0 = table_v[ch, pl.ds(0, _L)]
            run1 = table_v[ch, pl.ds(_L, _L)]
            xj0 = run0
            xj1 = run1
            for j in range(_MC - 2, 0, -1):
                ch = cvs[j][n]
                run0 = run0 + table_v[ch, pl.ds(0, _L)]
                run1 = run1 + table_v[ch, pl.ds(_L, _L)]
                xj0 = xj0 + run0
                xj1 = xj1 + run1
            ch = cvs[0][n]
            r00 = table_v[ch, pl.ds(0, _L)]
            r01 = table_v[ch, pl.ds(_L, _L)]
            run0 = run0 + r00
            run1 = run1 + r01
            cond = jnp.broadcast_to(m1, (_L,)) > 0.5
            xr0 = jnp.where(cond, 0.5 * r00, rec * xj0)
            xr1 = jnp.where(cond, 0.5 * r01, rec * xj1)
            xs_v[node, pl.ds(0, _L)] = xr0
            xs_v[node, pl.ds(_L, _L)] = xr1
            xj_v[node, pl.ds(0, _L)] = run0 - xr0
            xj_v[node, pl.ds(_L, _L)] = run1 - xr1
        return carry

    lax.fori_loop(0, _NODES_W // _L, group_body, 0)
    pltpu.sync_copy(xs_v, xs_hbm.at[pl.ds(ng * _NODES_W, _NODES_W), pl.ds(cb * _CB, _CB)])
    pltpu.sync_copy(xj_v, xj_hbm.at[pl.ds(ng * _NODES_W, _NODES_W), pl.ds(cb * _CB, _CB)])


@jax.jit
def _sc_gather(table, ch_t):
    mesh = plsc.VectorSubcoreMesh(core_axis_name="c", subcore_axis_name="s")
    f = pl.kernel(
        _sc_body,
        out_type=[jax.ShapeDtypeStruct((_BN, _E), jnp.float32),
                  jax.ShapeDtypeStruct((_BN, _E), jnp.float32)],
        mesh=mesh,
        scratch_types=[
            pltpu.VMEM((_N, _CB), jnp.float32),
            pltpu.VMEM((_MC, _NODES_W), jnp.int32),
            pltpu.VMEM((_NODES_W, _CB), jnp.float32),
            pltpu.VMEM((_NODES_W, _CB), jnp.float32),
        ],
        compiler_params=pltpu.CompilerParams(use_tc_tiling_on_sc=False,
                                             needs_layout_passes=False),
    )
    return f(table, ch_t)


def _tc_body(nodes_ref, xr_ref, xl_ref, a_ref, b1_ref,
             w2_ref, b2_ref, g_ref, bfc_ref, out_ref):
    x = jnp.dot(nodes_ref[...], a_ref[0], preferred_element_type=jnp.float32)
    x = x + jnp.dot(xr_ref[...], a_ref[1], preferred_element_type=jnp.float32)
    x = x + jnp.dot(xl_ref[...], a_ref[2], preferred_element_type=jnp.float32)
    h1 = jnp.tanh(x + b1_ref[...])
    h2 = jnp.tanh(
        lax.dot_general(h1, w2_ref[...], (((1,), (1,)), ((), ())),
                        preferred_element_type=jnp.float32) + b2_ref[...])
    m8 = jnp.max(h2.reshape(8, _N // 8, _C2), axis=1)
    m4 = jnp.max(m8.reshape(4, 2, _C2), axis=1)
    m2 = jnp.max(m4.reshape(2, 2, _C2), axis=1)
    m1 = jnp.max(m2, axis=0, keepdims=True)
    p = jnp.concatenate([m1, m2, m4, m8], axis=0)
    o = jnp.sum(p[:, :, None] * g_ref[...], axis=(0, 1)) + bfc_ref[0]
    out_ref[pl.ds(pl.program_id(0), 1), :] = o[None, :]


@functools.partial(jax.jit, static_argnames=())
def _tc_dense(nodes_f, xr_t, xl_t, a, b1, w2, b2, g, bfc):
    full = lambda shape: pl.BlockSpec(shape, lambda b: (0,) * len(shape))
    return pl.pallas_call(
        _tc_body,
        grid=(_B,),
        in_specs=[
            pl.BlockSpec((_N, _E), lambda b: (b, 0)),
            pl.BlockSpec((_N, _E), lambda b: (b, 0)),
            pl.BlockSpec((_N, _E), lambda b: (b, 0)),
            full((3, _E, _C1)),
            full((1, _C1)), full((_C2, _C1)), full((1, _C2)),
            full((15, _C2, _LBL)), full((1, _LBL)),
        ],
        out_specs=pl.BlockSpec((_B, _LBL), lambda b: (0, 0)),
        out_shape=jax.ShapeDtypeStruct((_B, _LBL), jnp.float32),
    )(nodes_f, xr_t, xl_t, a, b1, w2, b2, g, bfc)


def kernel(nodes, children, W1, b1, W2, b2, Wfc, bfc):
    xr_t, xl_t = _sc_gather(nodes[0], children.reshape(_BN, _MC).T)

    a = W1.reshape(_C1, _E, 3).transpose(2, 1, 0)
    g1 = Wfc[:, 0:120].reshape(_LBL, _C2, 1).transpose(2, 1, 0)
    g2 = Wfc[:, 120:360].reshape(_LBL, _C2, 2).transpose(2, 1, 0)
    g3 = Wfc[:, 360:840].reshape(_LBL, _C2, 4).transpose(2, 1, 0)
    g4 = Wfc[:, 840:1800].reshape(_LBL, _C2, 8).transpose(2, 1, 0)
    g = jnp.concatenate([g1, g2, g3, g4], axis=0)
    return _tc_dense(nodes.reshape(_BN, _E), xr_t, xl_t, a,
                     b1[None, :], W2, b2[None, :], g, bfc[None, :])


# speedup vs baseline: 1.5027x; 1.0082x over previous
import functools

import jax
import jax.numpy as jnp
from jax import lax
from jax.experimental import pallas as pl
from jax.experimental.pallas import tpu as pltpu
from jax.experimental.pallas import tpu_sc as plsc

_B, _N, _E, _MC = 8, 1024, 128, 16
_BN = _B * _N
_C1, _C2, _LBL = 240, 120, 104
_NW = 32
_NGRP = 8
_NCB = 4
_CB = _E // _NCB
_NODES_W = _BN // _NGRP
_L = 16
_UN = 16


def _sc_body(table_hbm, ch_hbm, xs_hbm, xj_hbm, table_v, ch_v, xs_v, xj_v):
    wid = lax.axis_index("s") * 2 + lax.axis_index("c")
    ng = wid // _NCB
    cb = wid % _NCB
    pltpu.sync_copy(table_hbm.at[:, pl.ds(cb * _CB, _CB)], table_v)
    pltpu.sync_copy(ch_hbm.at[:, pl.ds(ng * _NODES_W, _NODES_W)], ch_v)
    zero = jnp.zeros((_L,), jnp.float32)
    table_v[0, pl.ds(0, _L)] = zero
    table_v[0, pl.ds(_L, _L)] = zero

    one = jnp.full((_L,), 1.0, jnp.float32)
    izero = jnp.zeros((_L,), jnp.int32)

    def group_body(g, carry):
        base = g * _L
        cvs = [ch_v[j, pl.ds(base, _L)] for j in range(_MC)]
        ns = zero
        for j in range(_MC):
            ns = ns + jnp.where(cvs[j] != izero, one, zero)
        recv = one / (ns - one)
        m1v = jnp.where(ns == one, one, zero)
        for n in range(_L):
            node = base + n
            rec = recv[n]
            m1 = m1v[n]
            ch = cvs[_MC - 1][n]
            run0 = table_v[ch, pl.ds(0, _L)]
            run1 = table_v[ch, pl.ds(_L, _L)]
            xj0 = run0
            xj1 = run1
            for j in range(_MC - 2, 0, -1):
                ch = cvs[j][n]
                run0 = run0 + table_v[ch, pl.ds(0, _L)]
                run1 = run1 + table_v[ch, pl.ds(_L, _L)]
                xj0 = xj0 + run0
                xj1 = xj1 + run1
            ch = cvs[0][n]
            r00 = table_v[ch, pl.ds(0, _L)]
            r01 = table_v[ch, pl.ds(_L, _L)]
            run0 = run0 + r00
            run1 = run1 + r01
            cond = jnp.broadcast_to(m1, (_L,)) > 0.5
            xr0 = jnp.where(cond, 0.5 * r00, rec * xj0)
            xr1 = jnp.where(cond, 0.5 * r01, rec * xj1)
            xs_v[node, pl.ds(0, _L)] = xr0
            xs_v[node, pl.ds(_L, _L)] = xr1
            xj_v[node, pl.ds(0, _L)] = run0
            xj_v[node, pl.ds(_L, _L)] = run1
        return carry

    lax.fori_loop(0, _NODES_W // _L, group_body, 0)
    pltpu.sync_copy(xs_v, xs_hbm.at[pl.ds(ng * _NODES_W, _NODES_W), pl.ds(cb * _CB, _CB)])
    pltpu.sync_copy(xj_v, xj_hbm.at[pl.ds(ng * _NODES_W, _NODES_W), pl.ds(cb * _CB, _CB)])


@jax.jit
def _sc_gather(table, ch_t):
    mesh = plsc.VectorSubcoreMesh(core_axis_name="c", subcore_axis_name="s")
    f = pl.kernel(
        _sc_body,
        out_type=[jax.ShapeDtypeStruct((_BN, _E), jnp.float32),
                  jax.ShapeDtypeStruct((_BN, _E), jnp.float32)],
        mesh=mesh,
        scratch_types=[
            pltpu.VMEM((_N, _CB), jnp.float32),
            pltpu.VMEM((_MC, _NODES_W), jnp.int32),
            pltpu.VMEM((_NODES_W, _CB), jnp.float32),
            pltpu.VMEM((_NODES_W, _CB), jnp.float32),
        ],
        compiler_params=pltpu.CompilerParams(use_tc_tiling_on_sc=False,
                                             needs_layout_passes=False),
    )
    return f(table, ch_t)


def _tc_body(nodes_ref, xr_ref, xl_ref, a_ref, b1_ref,
             w2_ref, b2_ref, g_ref, bfc_ref, out_ref):
    x = jnp.dot(nodes_ref[...], a_ref[0], preferred_element_type=jnp.float32)
    x = x + jnp.dot(xr_ref[...], a_ref[1], preferred_element_type=jnp.float32)
    x = x + jnp.dot(xl_ref[...], a_ref[2], preferred_element_type=jnp.float32)
    h1 = jnp.tanh(x + b1_ref[...])
    h2 = jnp.tanh(
        lax.dot_general(h1, w2_ref[...], (((1,), (1,)), ((), ())),
                        preferred_element_type=jnp.float32) + b2_ref[...])
    m8 = jnp.max(h2.reshape(8, _N // 8, _C2), axis=1)
    m4 = jnp.max(m8.reshape(4, 2, _C2), axis=1)
    m2 = jnp.max(m4.reshape(2, 2, _C2), axis=1)
    m1 = jnp.max(m2, axis=0, keepdims=True)
    p = jnp.concatenate([m1, m2, m4, m8], axis=0)
    o = jnp.sum(p[:, :, None] * g_ref[...], axis=(0, 1)) + bfc_ref[0]
    out_ref[pl.ds(pl.program_id(0), 1), :] = o[None, :]


@functools.partial(jax.jit, static_argnames=())
def _tc_dense(nodes_f, xr_t, xl_t, a, b1, w2, b2, g, bfc):
    full = lambda shape: pl.BlockSpec(shape, lambda b: (0,) * len(shape))
    return pl.pallas_call(
        _tc_body,
        grid=(_B,),
        in_specs=[
            pl.BlockSpec((_N, _E), lambda b: (b, 0)),
            pl.BlockSpec((_N, _E), lambda b: (b, 0)),
            pl.BlockSpec((_N, _E), lambda b: (b, 0)),
            full((3, _E, _C1)),
            full((1, _C1)), full((_C2, _C1)), full((1, _C2)),
            full((15, _C2, _LBL)), full((1, _LBL)),
        ],
        out_specs=pl.BlockSpec((_B, _LBL), lambda b: (0, 0)),
        out_shape=jax.ShapeDtypeStruct((_B, _LBL), jnp.float32),
    )(nodes_f, xr_t, xl_t, a, b1, w2, b2, g, bfc)


def kernel(nodes, children, W1, b1, W2, b2, Wfc, bfc):
    xr_t, xl_t = _sc_gather(nodes[0], children.reshape(_BN, _MC).T)

    a = W1.reshape(_C1, _E, 3).transpose(2, 1, 0)
    a = jnp.stack([a[0], a[1] - a[2], a[2]])
    g1 = Wfc[:, 0:120].reshape(_LBL, _C2, 1).transpose(2, 1, 0)
    g2 = Wfc[:, 120:360].reshape(_LBL, _C2, 2).transpose(2, 1, 0)
    g3 = Wfc[:, 360:840].reshape(_LBL, _C2, 4).transpose(2, 1, 0)
    g4 = Wfc[:, 840:1800].reshape(_LBL, _C2, 8).transpose(2, 1, 0)
    g = jnp.concatenate([g1, g2, g3, g4], axis=0)
    return _tc_dense(nodes.reshape(_BN, _E), xr_t, xl_t, a,
                     b1[None, :], W2, b2[None, :], g, bfc[None, :])


# speedup vs baseline: 1.5184x; 1.0104x over previous
import functools

import jax
import jax.numpy as jnp
from jax import lax
from jax.experimental import pallas as pl
from jax.experimental.pallas import tpu as pltpu
from jax.experimental.pallas import tpu_sc as plsc

_B, _N, _E, _MC = 8, 1024, 128, 16
_BN = _B * _N
_C1, _C2, _LBL = 240, 120, 104
_NW = 32
_NGRP = 8
_NCB = 4
_CB = _E // _NCB
_NODES_W = _BN // _NGRP
_L = 16


def _sc_body(table_hbm, ch_hbm, xr_hbm, xs_hbm, table_v, ch_v, xr_v, xs_v):
    wid = lax.axis_index("s") * 2 + lax.axis_index("c")
    ng = wid // _NCB
    cb = wid % _NCB
    pltpu.sync_copy(table_hbm.at[:, pl.ds(cb * _CB, _CB)], table_v)
    pltpu.sync_copy(ch_hbm.at[:, pl.ds(ng * _NODES_W, _NODES_W)], ch_v)

    def group(g, carry):
        base = g * _L
        one = jnp.full((_L,), 1.0, jnp.float32)
        zero = jnp.zeros((_L,), jnp.float32)
        izero = jnp.zeros((_L,), jnp.int32)
        cvs = [ch_v[j, pl.ds(base, _L)] for j in range(_MC)]
        ns = zero
        for j in range(_MC):
            ns = ns + jnp.where(cvs[j] != izero, one, zero)
        recv = one / (ns - one)
        m1v = jnp.where(ns == one, one, zero)
        for n in range(_L):
            node = base + n
            rec = recv[n]
            m1 = m1v[n]
            ch = cvs[_MC - 1][n]
            run0 = table_v[ch, pl.ds(0, _L)]
            run1 = table_v[ch, pl.ds(_L, _L)]
            xj0 = run0
            xj1 = run1
            for j in range(_MC - 2, 0, -1):
                ch = cvs[j][n]
                run0 = run0 + table_v[ch, pl.ds(0, _L)]
                run1 = run1 + table_v[ch, pl.ds(_L, _L)]
                xj0 = xj0 + run0
                xj1 = xj1 + run1
            ch = cvs[0][n]
            r00 = table_v[ch, pl.ds(0, _L)]
            r01 = table_v[ch, pl.ds(_L, _L)]
            run0 = run0 + r00
            run1 = run1 + r01
            cond = jnp.broadcast_to(m1, (_L,)) > 0.5
            xr0 = jnp.where(cond, 0.5 * r00, rec * xj0)
            xr1 = jnp.where(cond, 0.5 * r01, rec * xj1)
            xs_v[node, pl.ds(0, _L)] = run0
            xs_v[node, pl.ds(_L, _L)] = run1
            xr_v[node, pl.ds(0, _L)] = xr0
            xr_v[node, pl.ds(_L, _L)] = xr1
        return carry

    lax.fori_loop(0, _NODES_W // _L, group, 0)
    pltpu.sync_copy(xr_v, xr_hbm.at[pl.ds(ng * _NODES_W, _NODES_W), pl.ds(cb * _CB, _CB)])
    pltpu.sync_copy(xs_v, xs_hbm.at[pl.ds(ng * _NODES_W, _NODES_W), pl.ds(cb * _CB, _CB)])


@jax.jit
def _sc_gather(table, ch_t):
    mesh = plsc.VectorSubcoreMesh(core_axis_name="c", subcore_axis_name="s")
    f = pl.kernel(
        _sc_body,
        out_type=[jax.ShapeDtypeStruct((_BN, _E), jnp.float32),
                  jax.ShapeDtypeStruct((_BN, _E), jnp.float32)],
        mesh=mesh,
        scratch_types=[
            pltpu.VMEM((_N, _CB), jnp.float32),
            pltpu.VMEM((_MC, _NODES_W), jnp.int32),
            pltpu.VMEM((_NODES_W, _CB), jnp.float32),
            pltpu.VMEM((_NODES_W, _CB), jnp.float32),
        ],
        compiler_params=pltpu.CompilerParams(use_tc_tiling_on_sc=False,
                                             needs_layout_passes=False),
    )
    return f(table, ch_t)


def _tc_body(nodes_ref, xr_ref, xs_ref, a0_ref, a1_ref, a2_ref, b1_ref,
             w2_ref, b2_ref, g_ref, bfc_ref, out_ref):
    x = jnp.dot(nodes_ref[...], a0_ref[...], preferred_element_type=jnp.float32)
    x = x + jnp.dot(xr_ref[...], a1_ref[...], preferred_element_type=jnp.float32)
    x = x + jnp.dot(xs_ref[...], a2_ref[...], preferred_element_type=jnp.float32)
    h1 = jnp.tanh(x + b1_ref[...])
    h2 = jnp.tanh(jnp.dot(h1, w2_ref[...], preferred_element_type=jnp.float32)
                  + b2_ref[...])
    m8 = jnp.max(h2.reshape(8, _N // 8, _C2), axis=1)
    m4 = jnp.max(m8.reshape(4, 2, _C2), axis=1)
    m2 = jnp.max(m4.reshape(2, 2, _C2), axis=1)
    m1 = jnp.max(m2, axis=0, keepdims=True)
    p = jnp.concatenate([m1, m2, m4, m8], axis=0)
    o = jnp.sum(p[:, :, None] * g_ref[...], axis=(0, 1)) + bfc_ref[0]
    out_ref[pl.ds(pl.program_id(0), 1), :] = o[None, :]


@functools.partial(jax.jit, static_argnames=())
def _tc_dense(nodes_f, xr_t, xs_t, a0, a1, a2, b1, w2t, b2, g, bfc):
    full = lambda shape: pl.BlockSpec(shape, lambda b: (0,) * len(shape))
    return pl.pallas_call(
        _tc_body,
        grid=(_B,),
        in_specs=[
            pl.BlockSpec((_N, _E), lambda b: (b, 0)),
            pl.BlockSpec((_N, _E), lambda b: (b, 0)),
            pl.BlockSpec((_N, _E), lambda b: (b, 0)),
            full((_E, _C1)), full((_E, _C1)), full((_E, _C1)),
            full((1, _C1)), full((_C1, _C2)), full((1, _C2)),
            full((15, _C2, _LBL)), full((1, _LBL)),
        ],
        out_specs=pl.BlockSpec((_B, _LBL), lambda b: (0, 0)),
        out_shape=jax.ShapeDtypeStruct((_B, _LBL), jnp.float32),
    )(nodes_f, xr_t, xs_t, a0, a1, a2, b1, w2t, b2, g, bfc)


def kernel(nodes, children, W1, b1, W2, b2, Wfc, bfc):
    table = jnp.concatenate(
        [jnp.zeros((1, _E), nodes.dtype), nodes[0, 1:, :]], axis=0)
    ch_t = children.reshape(_BN, _MC).T
    xr_t, xs_t = _sc_gather(table, ch_t)

    a0 = W1[:, 0::3].T
    a1 = (W1[:, 1::3] - W1[:, 2::3]).T
    a2 = W1[:, 2::3].T
    g1 = Wfc[:, 0:120].reshape(_LBL, _C2, 1).transpose(2, 1, 0)
    g2 = Wfc[:, 120:360].reshape(_LBL, _C2, 2).transpose(2, 1, 0)
    g3 = Wfc[:, 360:840].reshape(_LBL, _C2, 4).transpose(2, 1, 0)
    g4 = Wfc[:, 840:1800].reshape(_LBL, _C2, 8).transpose(2, 1, 0)
    g = jnp.concatenate([g1, g2, g3, g4], axis=0)
    return _tc_dense(nodes.reshape(_BN, _E), xr_t, xs_t, a0, a1, a2,
                     b1[None, :], W2.T, b2[None, :], g, bfc[None, :])
